# baseline (speedup 1.0000x reference)
import functools

import jax
import jax.numpy as jnp
from jax.experimental import pallas as pl
from jax.experimental.pallas import tpu as pltpu

N = 10000
G = 64
D_IN = 256
HIDDEN = 256
NUM_CLASSES = 10

BR = 2000
NB = N // BR
EPS = 1e-5


def _fused_kernel(x_ref, batch_ref, w1_ref, g1_ref, b1_ref, w2_ref, g2_ref,
                  b2_ref, wfc_ref, bfc_ref, wfc1_ref, bfc1_ref, out_ref,
                  h1_ref, h2_ref, sum1_ref, sq1_ref, sum2_ref, sq2_ref,
                  pooled_ref, cnt_ref):
    pid = pl.program_id(0)
    fN = jnp.float32(N)

    @pl.when(pid < NB)
    def _phase_a():
        j = pid
        h = jnp.dot(x_ref[...], w1_ref[...],
                    preferred_element_type=jnp.float32)
        h1_ref[pl.ds(j * BR, BR), :] = h
        s = jnp.sum(h, axis=0, keepdims=True)
        q = jnp.sum(h * h, axis=0, keepdims=True)

        @pl.when(j == 0)
        def _():
            sum1_ref[...] = s
            sq1_ref[...] = q

        @pl.when(j > 0)
        def _():
            sum1_ref[...] += s
            sq1_ref[...] += q

    @pl.when((pid >= NB) & (pid < 2 * NB))
    def _phase_b():
        j = pid - NB
        mu = sum1_ref[...] / fN
        var = sq1_ref[...] / fN - mu * mu
        a = g1_ref[...] * jax.lax.rsqrt(var + EPS)
        c = b1_ref[...] - mu * a
        hb = h1_ref[pl.ds(j * BR, BR), :]
        hn = jnp.maximum(hb * a + c, 0.0).astype(jnp.bfloat16)
        h2 = jnp.dot(hn, w2_ref[...], preferred_element_type=jnp.float32)
        h2_ref[pl.ds(j * BR, BR), :] = h2
        s = jnp.sum(h2, axis=0, keepdims=True)
        q = jnp.sum(h2 * h2, axis=0, keepdims=True)

        @pl.when(j == 0)
        def _():
            sum2_ref[...] = s
            sq2_ref[...] = q

        @pl.when(j > 0)
        def _():
            sum2_ref[...] += s
            sq2_ref[...] += q

    @pl.when((pid >= 2 * NB) & (pid < 3 * NB))
    def _phase_c():
        j = pid - 2 * NB
        mu = sum2_ref[...] / fN
        var = sq2_ref[...] / fN - mu * mu
        a = g2_ref[...] * jax.lax.rsqrt(var + EPS)
        c = b2_ref[...] - mu * a
        hb = h2_ref[pl.ds(j * BR, BR), :]
        hn = jnp.maximum(hb * a + c, 0.0).astype(jnp.bfloat16)
        oh = (batch_ref[...] ==
              jax.lax.broadcasted_iota(jnp.int32, (BR, G), 1)
              ).astype(jnp.bfloat16)
        dn = (((0,), (0,)), ((), ()))
        pb = jax.lax.dot_general(oh, hn, dn,
                                 preferred_element_type=jnp.float32)
        cb = jax.lax.dot_general(oh, jnp.ones((BR, 8), jnp.bfloat16), dn,
                                 preferred_element_type=jnp.float32)

        @pl.when(j == 0)
        def _():
            pooled_ref[...] = pb
            cnt_ref[...] = cb

        @pl.when(j > 0)
        def _():
            pooled_ref[...] += pb
            cnt_ref[...] += cb

    @pl.when(pid == 3 * NB)
    def _phase_d():
        cnt = jnp.maximum(cnt_ref[:, 0:1], 1.0)
        pooled = pooled_ref[...] / cnt
        h3 = jnp.maximum(
            jnp.dot(pooled, wfc_ref[...], preferred_element_type=jnp.float32)
            + bfc_ref[...], 0.0)
        logits = jnp.dot(h3, wfc1_ref[...],
                         preferred_element_type=jnp.float32) + bfc1_ref[...]
        m = jnp.max(logits, axis=-1, keepdims=True)
        sh = logits - m
        lse = jnp.log(jnp.sum(jnp.exp(sh), axis=-1, keepdims=True))
        out_ref[...] = sh - lse


@functools.partial(jax.jit, static_argnames=("interpret",))
def _run(x, batch, Wg1, g1, b1, Wg2, g2, b2, Wfc, bfc, Wfc1, bfc1,
         interpret=False):
    x = x.astype(jnp.bfloat16)
    Wg1 = Wg1.astype(jnp.bfloat16)
    Wg2 = Wg2.astype(jnp.bfloat16)
    batch2d = batch.reshape(N, 1)
    grid = (3 * NB + 1,)
    row = lambda r: (r, 0)
    const = lambda i: (0, 0)
    x_map = lambda i: row(jnp.where(i < NB, i, NB - 1))
    b_map = lambda i: row(jnp.clip(i - 2 * NB, 0, NB - 1))
    in_specs = [
        pl.BlockSpec((BR, D_IN), x_map),
        pl.BlockSpec((BR, 1), b_map),
        pl.BlockSpec((D_IN, HIDDEN), const),
        pl.BlockSpec((1, HIDDEN), const),
        pl.BlockSpec((1, HIDDEN), const),
        pl.BlockSpec((HIDDEN, HIDDEN), const),
        pl.BlockSpec((1, HIDDEN), const),
        pl.BlockSpec((1, HIDDEN), const),
        pl.BlockSpec((HIDDEN, HIDDEN), const),
        pl.BlockSpec((1, HIDDEN), const),
        pl.BlockSpec((HIDDEN, NUM_CLASSES), const),
        pl.BlockSpec((1, NUM_CLASSES), const),
    ]
    out = pl.pallas_call(
        _fused_kernel,
        grid=grid,
        in_specs=in_specs,
        out_specs=pl.BlockSpec((G, NUM_CLASSES), const),
        out_shape=jax.ShapeDtypeStruct((G, NUM_CLASSES), jnp.float32),
        scratch_shapes=[
            pltpu.VMEM((N, HIDDEN), jnp.float32),
            pltpu.VMEM((N, HIDDEN), jnp.float32),
            pltpu.VMEM((1, HIDDEN), jnp.float32),
            pltpu.VMEM((1, HIDDEN), jnp.float32),
            pltpu.VMEM((1, HIDDEN), jnp.float32),
            pltpu.VMEM((1, HIDDEN), jnp.float32),
            pltpu.VMEM((G, HIDDEN), jnp.float32),
            pltpu.VMEM((G, 8), jnp.float32),
        ],
        interpret=interpret,
    )(x, batch2d, Wg1, g1.reshape(1, HIDDEN), b1.reshape(1, HIDDEN),
      Wg2, g2.reshape(1, HIDDEN), b2.reshape(1, HIDDEN),
      Wfc, bfc.reshape(1, HIDDEN), Wfc1, bfc1.reshape(1, NUM_CLASSES))
    return out


def kernel(x, edge_index, batch, Wg1, bg1, g1, b1, Wg2, bg2, g2, b2,
           Wfc, bfc, Wfc1, bfc1):
    del edge_index, bg1, bg2
    return _run(x, batch, Wg1, g1, b1, Wg2, g2, b2, Wfc, bfc, Wfc1, bfc1)


# speedup vs baseline: 1.2663x; 1.2663x over previous
import functools

import jax
import jax.numpy as jnp
from jax.experimental import pallas as pl
from jax.experimental.pallas import tpu as pltpu

N = 10000
G = 64
D_IN = 256
HIDDEN = 256
NUM_CLASSES = 10
EPS = 1e-5

_DN_ROWS = (((0,), (0,)), ((), ()))


def _colsum(v, ones8):
    return jax.lax.dot_general(ones8, v, _DN_ROWS,
                               preferred_element_type=jnp.float32)[0:1, :]


def _fused_kernel(x_ref, batch_ref, w1_ref, g1_ref, b1_ref, w2_ref, g2_ref,
                  b2_ref, wfc_ref, bfc_ref, wfc1_ref, bfc1_ref, out_ref):
    fN = jnp.float32(N)
    ones8 = jnp.ones((N, 8), jnp.float32)

    h1 = jnp.dot(x_ref[...], w1_ref[...], preferred_element_type=jnp.float32)
    mu1 = _colsum(h1, ones8) / fN
    var1 = _colsum(h1 * h1, ones8) / fN - mu1 * mu1
    a1 = g1_ref[...] * jax.lax.rsqrt(var1 + EPS)
    c1 = b1_ref[...] - mu1 * a1
    hn1 = jnp.maximum(h1 * a1 + c1, 0.0)

    h2 = jnp.dot(hn1, w2_ref[...], preferred_element_type=jnp.float32)
    mu2 = _colsum(h2, ones8) / fN
    var2 = _colsum(h2 * h2, ones8) / fN - mu2 * mu2
    a2 = g2_ref[...] * jax.lax.rsqrt(var2 + EPS)
    c2 = b2_ref[...] - mu2 * a2
    hn2 = jnp.maximum(h2 * a2 + c2, 0.0)

    oh = (batch_ref[...] ==
          jax.lax.broadcasted_iota(jnp.int32, (N, G), 1)).astype(jnp.float32)
    sums = jax.lax.dot_general(oh, hn2, _DN_ROWS,
                               preferred_element_type=jnp.float32)
    cnt = jax.lax.dot_general(oh, ones8, _DN_ROWS,
                              preferred_element_type=jnp.float32)[:, 0:1]
    pooled = sums / jnp.maximum(cnt, 1.0)

    h3 = jnp.maximum(
        jnp.dot(pooled, wfc_ref[...], preferred_element_type=jnp.float32)
        + bfc_ref[...], 0.0)
    logits = jnp.dot(h3, wfc1_ref[...],
                     preferred_element_type=jnp.float32) + bfc1_ref[...]
    m = jnp.max(logits, axis=-1, keepdims=True)
    sh = logits - m
    lse = jnp.log(jnp.sum(jnp.exp(sh), axis=-1, keepdims=True))
    out_ref[...] = sh - lse


@functools.partial(jax.jit, static_argnames=("interpret",))
def _run(x, batch, Wg1, g1, b1, Wg2, g2, b2, Wfc, bfc, Wfc1, bfc1,
         interpret=False):
    batch2d = batch.reshape(N, 1)
    out = pl.pallas_call(
        _fused_kernel,
        out_shape=jax.ShapeDtypeStruct((G, NUM_CLASSES), jnp.float32),
        interpret=interpret,
    )(x, batch2d, Wg1, g1.reshape(1, HIDDEN), b1.reshape(1, HIDDEN),
      Wg2, g2.reshape(1, HIDDEN), b2.reshape(1, HIDDEN),
      Wfc, bfc.reshape(1, HIDDEN), Wfc1, bfc1.reshape(1, NUM_CLASSES))
    return out


def kernel(x, edge_index, batch, Wg1, bg1, g1, b1, Wg2, bg2, g2, b2,
           Wfc, bfc, Wfc1, bfc1):
    del edge_index, bg1, bg2
    return _run(x, batch, Wg1, g1, b1, Wg2, g2, b2, Wfc, bfc, Wfc1, bfc1)


# speedup vs baseline: 1.2901x; 1.0189x over previous
import functools

import jax
import jax.numpy as jnp
from jax.experimental import pallas as pl
from jax.experimental.pallas import tpu as pltpu

N = 10000
G = 64
D_IN = 256
HIDDEN = 256
NUM_CLASSES = 10
EPS = 1e-5

_DN_ROWS = (((0,), (0,)), ((), ()))


def _colsum(v, ones8):
    return jax.lax.dot_general(ones8, v, _DN_ROWS,
                               preferred_element_type=jnp.float32)[0:1, :]


def _fused_kernel(x_ref, batch_ref, w1_ref, g1_ref, b1_ref, w2_ref, g2_ref,
                  b2_ref, wfc_ref, bfc_ref, wfc1_ref, bfc1_ref, out_ref):
    fN = jnp.float32(N)
    bf16 = jnp.bfloat16
    ones8 = jnp.ones((N, 8), bf16)

    h1 = jnp.dot(x_ref[...].astype(bf16), w1_ref[...].astype(bf16),
                 preferred_element_type=jnp.float32)
    hb1 = h1.astype(bf16)
    mu1 = _colsum(hb1, ones8) / fN
    var1 = _colsum(hb1 * hb1, ones8) / fN - mu1 * mu1
    a1 = (g1_ref[...] * jax.lax.rsqrt(var1 + EPS)).astype(bf16)
    c1 = (b1_ref[...] - mu1 * (g1_ref[...] * jax.lax.rsqrt(var1 + EPS))
          ).astype(bf16)
    hn1 = jnp.maximum(hb1 * a1 + c1, bf16(0.0))

    h2 = jnp.dot(hn1, w2_ref[...].astype(bf16),
                 preferred_element_type=jnp.float32)
    hb2 = h2.astype(bf16)
    mu2 = _colsum(hb2, ones8) / fN
    var2 = _colsum(hb2 * hb2, ones8) / fN - mu2 * mu2
    a2f = g2_ref[...] * jax.lax.rsqrt(var2 + EPS)
    a2 = a2f.astype(bf16)
    c2 = (b2_ref[...] - mu2 * a2f).astype(bf16)
    hn2 = jnp.maximum(hb2 * a2 + c2, bf16(0.0))

    oh = (batch_ref[...] ==
          jax.lax.broadcasted_iota(jnp.int32, (N, G), 1)).astype(bf16)
    sums = jax.lax.dot_general(oh, hn2, _DN_ROWS,
                               preferred_element_type=jnp.float32)
    cnt = jax.lax.dot_general(oh, ones8, _DN_ROWS,
                              preferred_element_type=jnp.float32)[:, 0:1]
    pooled = sums / jnp.maximum(cnt, 1.0)

    h3 = jnp.maximum(
        jnp.dot(pooled, wfc_ref[...], preferred_element_type=jnp.float32)
        + bfc_ref[...], 0.0)
    logits = jnp.dot(h3, wfc1_ref[...],
                     preferred_element_type=jnp.float32) + bfc1_ref[...]
    m = jnp.max(logits, axis=-1, keepdims=True)
    sh = logits - m
    lse = jnp.log(jnp.sum(jnp.exp(sh), axis=-1, keepdims=True))
    out_ref[...] = sh - lse


@functools.partial(jax.jit, static_argnames=("interpret",))
def _run(x, batch, Wg1, g1, b1, Wg2, g2, b2, Wfc, bfc, Wfc1, bfc1,
         interpret=False):
    batch2d = batch.reshape(N, 1)
    out = pl.pallas_call(
        _fused_kernel,
        out_shape=jax.ShapeDtypeStruct((G, NUM_CLASSES), jnp.float32),
        interpret=interpret,
    )(x, batch2d, Wg1, g1.reshape(1, HIDDEN), b1.reshape(1, HIDDEN),
      Wg2, g2.reshape(1, HIDDEN), b2.reshape(1, HIDDEN),
      Wfc, bfc.reshape(1, HIDDEN), Wfc1, bfc1.reshape(1, NUM_CLASSES))
    return out


def kernel(x, edge_index, batch, Wg1, bg1, g1, b1, Wg2, bg2, g2, b2,
           Wfc, bfc, Wfc1, bfc1):
    del edge_index, bg1, bg2
    return _run(x, batch, Wg1, g1, b1, Wg2, g2, b2, Wfc, bfc, Wfc1, bfc1)


# speedup vs baseline: 1.7428x; 1.3509x over previous
import functools

import jax
import jax.numpy as jnp
from jax.experimental import pallas as pl

N = 10000
G = 64
D_IN = 256
HIDDEN = 256
NUM_CLASSES = 10
EPS = 1e-5

_DN_ROWS = (((0,), (0,)), ((), ()))


def _colsum(v, ones8):
    return jax.lax.dot_general(ones8, v, _DN_ROWS,
                               preferred_element_type=jnp.float32)[0:1, :]


def _fused_kernel(x_ref, batch_ref, w1_ref, g1_ref, b1_ref, w2_ref, g2_ref,
                  b2_ref, wfc_ref, bfc_ref, wfc1_ref, bfc1_ref, out_ref):
    fN = jnp.float32(N)
    bf16 = jnp.bfloat16
    ones8 = jnp.ones((N, 8), bf16)

    h1 = jnp.dot(x_ref[...].astype(bf16), w1_ref[...].astype(bf16),
                 preferred_element_type=jnp.float32)
    hb1 = h1.astype(bf16)
    mu1 = _colsum(hb1, ones8) / fN
    var1 = _colsum(hb1 * hb1, ones8) / fN - mu1 * mu1
    a1f = g1_ref[...] * jax.lax.rsqrt(var1 + EPS)
    a1 = a1f.astype(bf16)
    c1 = (b1_ref[...] - mu1 * a1f).astype(bf16)
    hn1 = jnp.maximum(hb1 * a1 + c1, bf16(0.0))

    h2 = jnp.dot(hn1, w2_ref[...].astype(bf16),
                 preferred_element_type=jnp.float32)
    hb2 = h2.astype(bf16)
    mu2 = _colsum(hb2, ones8) / fN
    var2 = _colsum(hb2 * hb2, ones8) / fN - mu2 * mu2
    a2f = g2_ref[...] * jax.lax.rsqrt(var2 + EPS)
    a2 = a2f.astype(bf16)
    c2 = (b2_ref[...] - mu2 * a2f).astype(bf16)
    hn2 = jnp.maximum(hb2 * a2 + c2, bf16(0.0))

    ohT = (batch_ref[...] ==
           jax.lax.broadcasted_iota(jnp.int32, (G, N), 0)).astype(bf16)
    dn_mm = (((1,), (0,)), ((), ()))
    sums = jax.lax.dot_general(ohT, hn2, dn_mm,
                               preferred_element_type=jnp.float32)
    cnt = jnp.sum(ohT.astype(jnp.float32), axis=1, keepdims=True)
    pooled = sums / jnp.maximum(cnt, 1.0)

    h3 = jnp.maximum(
        jnp.dot(pooled, wfc_ref[...], preferred_element_type=jnp.float32)
        + bfc_ref[...], 0.0)
    logits = jnp.dot(h3, wfc1_ref[...],
                     preferred_element_type=jnp.float32) + bfc1_ref[...]
    m = jnp.max(logits, axis=-1, keepdims=True)
    sh = logits - m
    lse = jnp.log(jnp.sum(jnp.exp(sh), axis=-1, keepdims=True))
    out_ref[...] = sh - lse


@functools.partial(jax.jit, static_argnames=("interpret",))
def _run(x, batch, Wg1, g1, b1, Wg2, g2, b2, Wfc, bfc, Wfc1, bfc1,
         interpret=False):
    batch2d = batch.reshape(1, N)
    out = pl.pallas_call(
        _fused_kernel,
        out_shape=jax.ShapeDtypeStruct((G, NUM_CLASSES), jnp.float32),
        interpret=interpret,
    )(x, batch2d, Wg1, g1.reshape(1, HIDDEN), b1.reshape(1, HIDDEN),
      Wg2, g2.reshape(1, HIDDEN), b2.reshape(1, HIDDEN),
      Wfc, bfc.reshape(1, HIDDEN), Wfc1, bfc1.reshape(1, NUM_CLASSES))
    return out


def kernel(x, edge_index, batch, Wg1, bg1, g1, b1, Wg2, bg2, g2, b2,
           Wfc, bfc, Wfc1, bfc1):
    del edge_index, bg1, bg2
    return _run(x, batch, Wg1, g1, b1, Wg2, g2, b2, Wfc, bfc, Wfc1, bfc1)


# speedup vs baseline: 2.6496x; 1.5203x over previous
import functools

import jax
import jax.numpy as jnp
from jax.experimental import pallas as pl

N = 10000
G = 64
D_IN = 256
HIDDEN = 256
NUM_CLASSES = 10
EPS = 1e-5

_DN_ROWS = (((0,), (0,)), ((), ()))


def _colsum_sq(v, ones8):
    return jax.lax.dot_general(ones8, v * v, _DN_ROWS,
                               preferred_element_type=jnp.float32)[0:1, :]


def _fused_kernel(x_ref, batch_ref, w1_ref, g1_ref, b1_ref, w2_ref, g2_ref,
                  b2_ref, wfc_ref, bfc_ref, wfc1t_ref, bfc1_ref, out_ref):
    fN = jnp.float32(N)
    bf16 = jnp.bfloat16
    ones8 = jnp.ones((N, 8), bf16)

    xb = x_ref[...].astype(bf16)
    w1 = w1_ref[...].astype(bf16)
    h1 = jnp.dot(xb, w1, preferred_element_type=jnp.float32)
    hb1 = h1.astype(bf16)
    csx = jnp.sum(x_ref[...], axis=0, keepdims=True)
    mu1 = jnp.dot(csx / fN, w1_ref[...], preferred_element_type=jnp.float32)
    var1 = _colsum_sq(hb1, ones8) / fN - mu1 * mu1
    a1f = g1_ref[...] * jax.lax.rsqrt(var1 + EPS)
    a1 = a1f.astype(bf16)
    c1 = (b1_ref[...] - mu1 * a1f).astype(bf16)
    hn1 = jnp.maximum(hb1 * a1 + c1, bf16(0.0))

    h2 = jnp.dot(hn1, w2_ref[...].astype(bf16),
                 preferred_element_type=jnp.float32)
    hb2 = h2.astype(bf16)
    cs1 = jnp.sum(hn1.astype(jnp.float32), axis=0, keepdims=True)
    mu2 = jnp.dot(cs1 / fN, w2_ref[...], preferred_element_type=jnp.float32)
    var2 = _colsum_sq(hb2, ones8) / fN - mu2 * mu2
    a2f = g2_ref[...] * jax.lax.rsqrt(var2 + EPS)
    a2 = a2f.astype(bf16)
    c2 = (b2_ref[...] - mu2 * a2f).astype(bf16)
    hn2 = jnp.maximum(hb2 * a2 + c2, bf16(0.0))

    b2d = batch_ref[...].reshape(1, N)
    ohT = (b2d == jax.lax.broadcasted_iota(jnp.int32, (G, N), 0)).astype(bf16)
    dn_mm = (((1,), (0,)), ((), ()))
    sums = jax.lax.dot_general(ohT, hn2, dn_mm,
                               preferred_element_type=jnp.float32)
    cnt = jnp.sum(ohT.astype(jnp.float32), axis=1, keepdims=True)
    pooled = sums / jnp.maximum(cnt, 1.0)

    h3 = jnp.maximum(
        jnp.dot(pooled, wfc_ref[...], preferred_element_type=jnp.float32)
        + bfc_ref[...], 0.0)
    dn_t = (((1,), (1,)), ((), ()))
    logits = jax.lax.dot_general(h3, wfc1t_ref[...], dn_t,
                                 preferred_element_type=jnp.float32)
    logits = logits + bfc1_ref[...]
    m = jnp.max(logits, axis=-1, keepdims=True)
    sh = logits - m
    lse = jnp.log(jnp.sum(jnp.exp(sh), axis=-1, keepdims=True))
    out_ref[...] = (sh - lse).T


@functools.partial(jax.jit, static_argnames=("interpret",))
def _run(x, batch, Wg1, g1, b1, Wg2, g2, b2, Wfc, bfc, Wfc1, bfc1,
         interpret=False):
    out_t = pl.pallas_call(
        _fused_kernel,
        out_shape=jax.ShapeDtypeStruct((NUM_CLASSES, G), jnp.float32),
        interpret=interpret,
    )(x, batch, Wg1, g1.reshape(1, HIDDEN), b1.reshape(1, HIDDEN),
      Wg2, g2.reshape(1, HIDDEN), b2.reshape(1, HIDDEN),
      Wfc, bfc.reshape(1, HIDDEN), Wfc1.T, bfc1.reshape(1, NUM_CLASSES))
    return out_t.T


def kernel(x, edge_index, batch, Wg1, bg1, g1, b1, Wg2, bg2, g2, b2,
           Wfc, bfc, Wfc1, bfc1):
    del edge_index, bg1, bg2
    return _run(x, batch, Wg1, g1, b1, Wg2, g2, b2, Wfc, bfc, Wfc1, bfc1)
